# baseline (speedup 1.0000x reference)
import functools

import jax
import jax.numpy as jnp
from jax import lax
from jax.experimental import pallas as pl
from jax.experimental.pallas import tpu as pltpu
from jax.experimental.pallas import tpu_sc as plsc

NUM_CORES = 2
NUM_SUBCORES = 16
NW = NUM_CORES * NUM_SUBCORES
LANES = 16
UNROLL = 16
TAIL = 32


@functools.lru_cache(maxsize=None)
def _make_gather_t(FD, V, B, D):
    rows_per_w = FD // NW
    HB = 4096
    nh = B // HB
    VT = V - TAIL
    VLO = (V // 2) // 128 * 128
    VHI = VT - VLO
    mesh = plsc.VectorSubcoreMesh(core_axis_name="c", subcore_axis_name="s")

    @functools.partial(
        pl.kernel,
        out_type=jax.ShapeDtypeStruct((FD, B), jnp.float32),
        mesh=mesh,
        scratch_types=[
            pltpu.VMEM((VLO,), jnp.float32),
            pltpu.VMEM((VHI,), jnp.float32),
            pltpu.VMEM((128,), jnp.float32),
            pltpu.VMEM((HB,), jnp.int32),
            pltpu.VMEM((HB,), jnp.int32),
            pltpu.VMEM((HB,), jnp.float32),
            pltpu.VMEM((HB,), jnp.float32),
            pltpu.SemaphoreType.DMA,
            pltpu.SemaphoreType.DMA,
            pltpu.SemaphoreType.DMA,
            pltpu.SemaphoreType.DMA,
            pltpu.SemaphoreType.DMA,
            pltpu.SemaphoreType.DMA,
        ],
        compiler_params=pltpu.CompilerParams(needs_layout_passes=False),
    )
    def k(tab_hbm, tails_hbm, idx_hbm, out_hbm,
          tab_lo, tab_hi, tails_v, idx_a, idx_b, out_a, out_b,
          s_lo, s_hi, s_tl, s_ia, s_ib, s_out):
        wid = lax.axis_index("s") * NUM_CORES + lax.axis_index("c")
        c0 = wid * rows_per_w
        iota = lax.iota(jnp.int32, LANES)
        idx_bufs = (idx_a, idx_b)
        idx_sems = (s_ia, s_ib)
        out_bufs = (out_a, out_b)

        def start_lo(c):
            pltpu.async_copy(tab_hbm.at[c, pl.ds(0, VLO)], tab_lo, s_lo)

        def wait_lo():
            pltpu.make_async_copy(tab_hbm.at[c0, pl.ds(0, VLO)],
                                  tab_lo, s_lo).wait()

        def start_hi(c):
            pltpu.async_copy(tab_hbm.at[c, pl.ds(VLO, VHI)], tab_hi, s_hi)
            pltpu.async_copy(tails_hbm.at[c], tails_v, s_tl)

        def wait_hi():
            pltpu.make_async_copy(tab_hbm.at[c0, pl.ds(VLO, VHI)],
                                  tab_hi, s_hi).wait()
            pltpu.make_async_copy(tails_hbm.at[c0], tails_v, s_tl).wait()

        def start_idx(f, h, slot):
            pltpu.async_copy(idx_hbm.at[f, pl.ds(h * HB, HB)],
                             idx_bufs[slot], idx_sems[slot])

        def wait_idx(slot):
            pltpu.make_async_copy(idx_hbm.at[0, pl.ds(0, HB)],
                                  idx_bufs[slot], idx_sems[slot]).wait()

        def start_out(c, h):
            pltpu.async_copy(out_bufs[h % 2],
                             out_hbm.at[c, pl.ds(h * HB, HB)], s_out)

        def wait_out():
            pltpu.make_async_copy(out_bufs[0],
                                  out_hbm.at[c0, pl.ds(0, HB)], s_out).wait()

        def pass1(h):
            buf = idx_bufs[h % 2]
            obuf = out_bufs[h % 2]

            def body(kk, carry, buf=buf, obuf=obuf):
                base = kk * (UNROLL * LANES)
                ivs = [buf[pl.ds(base + u * LANES, LANES)]
                       for u in range(UNROLL)]
                gs = [plsc.load_gather(tab_lo, [jnp.minimum(iv, VLO - 1)],
                                       mask=iv < VLO)
                      for iv in ivs]
                for u in range(UNROLL):
                    obuf[pl.ds(base + u * LANES, LANES)] = gs[u]
                return carry

            lax.fori_loop(0, HB // (UNROLL * LANES), body, 0)

        def pass2(h):
            buf = idx_bufs[h % 2]
            obuf = out_bufs[h % 2]

            def body(kk, carry, buf=buf, obuf=obuf, h=h):
                base = kk * (UNROLL * LANES)
                ivs = [buf[pl.ds(base + u * LANES, LANES)]
                       for u in range(UNROLL)]
                ms = [iv >= VLO for iv in ivs]
                gs = [plsc.load_gather(
                          tab_hi,
                          [jnp.minimum(jnp.maximum(iv - VLO, 0), VHI - 1)],
                          mask=m)
                      for iv, m in zip(ivs, ms)]
                mts = [iv >= VT for iv in ivs]
                gts = [plsc.load_gather(tails_v,
                                        [jnp.maximum(iv - VT, 0)], mask=mt)
                       for iv, mt in zip(ivs, mts)]
                for u in range(UNROLL):
                    g = jnp.where(mts[u], gts[u], gs[u])
                    plsc.store_scatter(obuf,
                                       [base + u * LANES + iota], g,
                                       mask=ms[u])
                return carry

            lax.fori_loop(0, HB // (UNROLL * LANES), body, 0)

        def row_body(j, carry):
            c = c0 + j
            f = c // D
            f_nxt = (c + 1) // D
            start_hi(c)
            wait_lo()
            @pl.when(j > 0)
            def _():
                wait_out()
            wait_idx(0)
            start_idx(f, 1, 1)
            pass1(0)
            @pl.when(j > 0)
            def _():
                wait_out()
            wait_idx(1)
            pass1(1)
            wait_hi()
            pass2(0)
            start_out(c, 0)
            start_idx(f, 2, 0)
            wait_out()
            wait_idx(0)
            pass1(2)
            pass2(1)
            start_out(c, 1)
            start_idx(f, 3, 1)
            wait_out()
            wait_idx(1)
            pass1(3)
            @pl.when(j < rows_per_w - 1)
            def _():
                start_lo(c + 1)
            pass2(2)
            start_out(c, 2)

            @pl.when(j < rows_per_w - 1)
            def _():
                start_idx(f_nxt, 0, 0)
            pass2(3)
            start_out(c, 3)
            return carry

        start_lo(c0)
        start_idx(c0 // D, 0, 0)
        lax.fori_loop(0, rows_per_w, row_body, 0)
        wait_out()
        wait_out()

    return k


def kernel(category_inputs, tables):
    B, F = category_inputs.shape
    _, V, D = tables.shape
    idx_t = category_inputs.astype(jnp.int32).T
    tab_t = jnp.transpose(tables, (0, 2, 1)).reshape(F * D, V)
    tails = jnp.pad(tab_t[:, V - TAIL:], ((0, 0), (0, 128 - TAIL)))
    out_t = _make_gather_t(F * D, V, B, D)(tab_t, tails, idx_t)
    return out_t.T


# speedup vs baseline: 1.4086x; 1.4086x over previous
import functools

import jax
import jax.numpy as jnp
from jax import lax
from jax.experimental import pallas as pl
from jax.experimental.pallas import tpu as pltpu
from jax.experimental.pallas import tpu_sc as plsc

NUM_CORES = 2
NUM_SUBCORES = 16
NW = NUM_CORES * NUM_SUBCORES
LANES = 16
UNROLL = 16


@functools.lru_cache(maxsize=None)
def _make_gather_t(FD, V, B, D):
    rows_per_w = FD // NW
    HB = 4096
    nh = B // HB
    mesh = plsc.VectorSubcoreMesh(core_axis_name="c", subcore_axis_name="s")

    @functools.partial(
        pl.kernel,
        out_type=jax.ShapeDtypeStruct((FD, B), jnp.float32),
        mesh=mesh,
        scratch_types=[
            pltpu.VMEM((V,), jnp.float32),
            pltpu.VMEM((HB,), jnp.int32),
            pltpu.VMEM((HB,), jnp.int32),
            pltpu.VMEM((HB,), jnp.float32),
            pltpu.VMEM((HB,), jnp.float32),
            pltpu.SemaphoreType.DMA,
            pltpu.SemaphoreType.DMA,
            pltpu.SemaphoreType.DMA,
            pltpu.SemaphoreType.DMA,
            pltpu.SemaphoreType.DMA,
        ],
        compiler_params=pltpu.CompilerParams(needs_layout_passes=False),
    )
    def k(tab_hbm, idx_hbm, out_hbm, tab_v, idx_a, idx_b, out_a, out_b,
          s_lo, s_hi, s_ia, s_ib, s_out):
        wid = lax.axis_index("s") * NUM_CORES + lax.axis_index("c")
        c0 = wid * rows_per_w
        idx_bufs = (idx_a, idx_b)
        idx_sems = (s_ia, s_ib)
        out_bufs = (out_a, out_b)

        def start_tab(c):
            pltpu.async_copy(tab_hbm.at[c], tab_v, s_lo)

        def wait_tab(c):
            pltpu.make_async_copy(tab_hbm.at[c], tab_v, s_lo).wait()

        def start_idx(f, h, slot):
            pltpu.async_copy(idx_hbm.at[f, pl.ds(h * HB, HB)],
                             idx_bufs[slot], idx_sems[slot])

        def wait_idx(slot):
            pltpu.make_async_copy(idx_hbm.at[0, pl.ds(0, HB)],
                                  idx_bufs[slot], idx_sems[slot]).wait()

        def wait_out(slot):
            pltpu.make_async_copy(out_bufs[slot],
                                  out_hbm.at[c0, pl.ds(0, HB)], s_out).wait()

        def row_body(j, carry):
            c = c0 + j
            f = c // D
            f_nxt = (c + 1) // D
            wait_tab(c)
            for h in range(nh):
                slot = h % 2
                if h + 1 < nh:
                    start_idx(f, h + 1, (h + 1) % 2)
                else:
                    @pl.when(j < rows_per_w - 1)
                    def _():
                        start_idx(f_nxt, 0, (h + 1) % 2)
                wait_idx(slot)
                if h < 2:
                    @pl.when(j > 0)
                    def _():
                        wait_out(slot)
                else:
                    wait_out(slot)
                buf = idx_bufs[slot]
                obuf = out_bufs[slot]

                def body(kk, carry3, buf=buf, obuf=obuf):
                    base = kk * (UNROLL * LANES)
                    ivs = [buf[pl.ds(base + u * LANES, LANES)]
                           for u in range(UNROLL)]
                    gs = [plsc.load_gather(tab_v, [iv]) for iv in ivs]
                    for u in range(UNROLL):
                        obuf[pl.ds(base + u * LANES, LANES)] = gs[u]
                    return carry3

                lax.fori_loop(0, HB // (UNROLL * LANES), body, 0)
                if h == nh - 1:
                    @pl.when(j < rows_per_w - 1)
                    def _():
                        start_tab(c + 1)
                pltpu.async_copy(obuf, out_hbm.at[c, pl.ds(h * HB, HB)], s_out)
            return carry

        start_tab(c0)
        start_idx(c0 // D, 0, 0)
        lax.fori_loop(0, rows_per_w, row_body, 0)
        wait_out(0)
        wait_out(1)

    return k


def kernel(category_inputs, tables):
    B, F = category_inputs.shape
    _, V, D = tables.shape
    idx_t = category_inputs.astype(jnp.int32).T
    tab_t = jnp.transpose(tables, (0, 2, 1)).reshape(F * D, V)
    out_t = _make_gather_t(F * D, V, B, D)(tab_t, idx_t)
    return out_t.T
